# Initial kernel scaffold; baseline (speedup 1.0000x reference)
import jax
import jax.numpy as jnp
from jax.experimental import pallas as pl


def kernel(x, weight):
    raise NotImplementedError("write your pallas kernel here")


# speedup vs baseline: 7.3088x; 7.3088x over previous
import functools

import jax
import jax.numpy as jnp
from jax.experimental import pallas as pl
from jax.experimental.pallas import tpu as pltpu


def _round_up(v, m):
    return (v + m - 1) // m * m


def _conv3_kernel(w_ref, x_ref, o_ref, *, l_out):
    xb = x_ref[...].astype(jnp.bfloat16)
    c, l = xb.shape
    zero_col = jnp.zeros((c, 1), jnp.bfloat16)
    x_prev = jnp.concatenate([zero_col, xb[:, : l - 1]], axis=1)
    x_next = jnp.concatenate([xb[:, 1:], zero_col], axis=1)
    acc = jnp.dot(w_ref[0], x_prev, preferred_element_type=jnp.float32)
    acc += jnp.dot(w_ref[1], xb, preferred_element_type=jnp.float32)
    acc += jnp.dot(w_ref[2], x_next, preferred_element_type=jnp.float32)
    o_ref[...] = jnp.maximum(acc, 0.0)


@jax.jit
def kernel(x, weight):
    N, C_in, L = x.shape
    C_out, C_in_w, K = weight.shape
    assert C_in_w == C_in and K == 3
    L_out = L

    C_in_pad = _round_up(C_in, 8)
    C_out_pad = _round_up(C_out, 8)
    L_pad = _round_up(L, 128)
    xp = jnp.pad(x, ((0, 0), (0, C_in_pad - C_in), (0, L_pad - L)))
    w3 = jnp.transpose(weight, (2, 0, 1)).astype(jnp.bfloat16)
    w3 = jnp.pad(w3, ((0, 0), (0, C_out_pad - C_out), (0, C_in_pad - C_in)))

    out = pl.pallas_call(
        functools.partial(_conv3_kernel, l_out=L_out),
        out_shape=jax.ShapeDtypeStruct((N, C_out_pad, L_pad), x.dtype),
        grid=(N,),
        in_specs=[
            pl.BlockSpec((K, C_out_pad, C_in_pad), lambda n: (0, 0, 0)),
            pl.BlockSpec((pl.Squeezed(), C_in_pad, L_pad), lambda n: (n, 0, 0)),
        ],
        out_specs=pl.BlockSpec((pl.Squeezed(), C_out_pad, L_pad),
                               lambda n: (n, 0, 0)),
        compiler_params=pltpu.CompilerParams(
            dimension_semantics=("parallel",),
        ),
    )(w3, xp)
    if C_out_pad != C_out or L_pad != L_out:
        out = out[:, :C_out, :L_out]
    return out
